# Initial kernel scaffold; baseline (speedup 1.0000x reference)
import jax
import jax.numpy as jnp
from jax.experimental import pallas as pl


def kernel(q_t, f4, f8, f16, f32, c_t, params):
    raise NotImplementedError("write your pallas kernel here")


# speedup vs baseline: 1.0073x; 1.0073x over previous
import functools

import jax
import jax.numpy as jnp
import numpy as np
from jax.experimental import pallas as pl
from jax.experimental.pallas import tpu as pltpu

K = 32
D = 128
NHEAD = 8
DH = D // NHEAD
H_IMG, W_IMG = 512, 512
STRIDE = 4
HF, WF = H_IMG // STRIDE, W_IMG // STRIDE
P = HF * WF
NUM_LEVEL = 4
N_POINTS = 4
N_LAYERS = 2
N_T = 512
DFF = 2 * D
SHAPES = [(HF, WF), (HF // 2, WF // 2), (HF // 4, WF // 4), (HF // 8, WF // 8)]
STARTS = [0, P, P + P // 4, P + P // 4 + P // 16]

ROWS_PER_BLK = 8


def _topk_body(c_ref, idx_ref, scratch):
    scratch[...] = c_ref[...]
    iota = jax.lax.broadcasted_iota(jnp.int32, (ROWS_PER_BLK, P), 1)
    kiota = jax.lax.broadcasted_iota(jnp.int32, (ROWS_PER_BLK, K), 1)

    def body(i, idxs):
        x = scratch[...]
        m = jnp.max(x, axis=1, keepdims=True)
        cand = jnp.where(x == m, iota, P)
        amin = jnp.min(cand, axis=1)
        scratch[...] = jnp.where(iota == amin[:, None], -jnp.inf, x)
        return jnp.where(kiota == i, amin[:, None], idxs)

    idxs = jax.lax.fori_loop(0, K, body, jnp.zeros((ROWS_PER_BLK, K), jnp.int32))
    idx_ref[0] = idxs


def _topk(c):
    nblk = N_T // ROWS_PER_BLK
    out = pl.pallas_call(
        _topk_body,
        grid=(nblk,),
        in_specs=[pl.BlockSpec((ROWS_PER_BLK, P), lambda i: (i, 0))],
        out_specs=pl.BlockSpec((1, ROWS_PER_BLK, K), lambda i: (i, 0, 0)),
        out_shape=jax.ShapeDtypeStruct((nblk, ROWS_PER_BLK, K), jnp.int32),
        scratch_shapes=[pltpu.VMEM((ROWS_PER_BLK, P), jnp.float32)],
    )(c)
    return out.reshape(N_T, K)


def _layer_norm(x, g, b):
    mu = jnp.mean(x, axis=-1, keepdims=True)
    var = jnp.var(x, axis=-1, keepdims=True)
    return (x - mu) / jnp.sqrt(var + 1e-5) * g + b


def _bilinear_gather(feat, Hl, Wl, x, y):
    x0 = jnp.floor(x); y0 = jnp.floor(y)
    wx = x - x0; wy = y - y0
    x0i = jnp.clip(x0.astype(jnp.int32), 0, Wl - 1)
    x1i = jnp.clip(x0i + 1, 0, Wl - 1)
    y0i = jnp.clip(y0.astype(jnp.int32), 0, Hl - 1)
    y1i = jnp.clip(y0i + 1, 0, Hl - 1)
    gather = jax.vmap(lambda f, i: f[i])
    v00 = gather(feat, y0i * Wl + x0i)
    v01 = gather(feat, y0i * Wl + x1i)
    v10 = gather(feat, y1i * Wl + x0i)
    v11 = gather(feat, y1i * Wl + x1i)
    return (v00 * ((1 - wx) * (1 - wy))[..., None] + v01 * (wx * (1 - wy))[..., None]
            + v10 * ((1 - wx) * wy)[..., None] + v11 * (wx * wy)[..., None])


def _dmsmha_block(lp, q, kv, ref):
    M = q.shape[0]
    v = (kv @ lp['W_val'] + lp['b_val']).reshape(-1, NHEAD, DH)
    off = (q @ lp['W_off'] + lp['b_off']).reshape(M, NHEAD, NUM_LEVEL, N_POINTS, 2)
    attn = (q @ lp['W_attn'] + lp['b_attn']).reshape(M, NHEAD, NUM_LEVEL * N_POINTS)
    attn = jax.nn.softmax(attn, axis=-1).reshape(M, NHEAD, NUM_LEVEL, N_POINTS)
    out = jnp.zeros((NHEAD, M, DH), jnp.float32)
    for l in range(NUM_LEVEL):
        Hl, Wl = SHAPES[l]
        s = STARTS[l]
        scale = jnp.array([Wl, Hl], jnp.float32)
        locs = ref[:, None, l, None, :] + off[:, :, l, :, :] / scale
        x = (locs[..., 0] * Wl - 0.5).transpose(1, 0, 2)
        y = (locs[..., 1] * Hl - 0.5).transpose(1, 0, 2)
        feat = v[s:s + Hl * Wl].transpose(1, 0, 2)
        sampled = _bilinear_gather(feat, Hl, Wl, x, y)
        w = attn[:, :, l, :].transpose(1, 0, 2)
        out = out + jnp.sum(sampled * w[..., None], axis=2)
    out = out.transpose(1, 0, 2).reshape(M, D) @ lp['W_out'] + lp['b_out']
    q = _layer_norm(q + out, lp['ln1_g'], lp['ln1_b'])
    ff = jax.nn.relu(q @ lp['ffn_w1'] + lp['ffn_b1']) @ lp['ffn_w2'] + lp['ffn_b2']
    return _layer_norm(q + ff, lp['ln2_g'], lp['ln2_b'])


def _mha_block(p, q, k, v):
    N = q.shape[0]
    qh = (q @ p['fus_Wq']).reshape(N, 1, NHEAD, DH).transpose(0, 2, 1, 3)
    kh = (k @ p['fus_Wk']).reshape(N, -1, NHEAD, DH).transpose(0, 2, 1, 3)
    vh = (v @ p['fus_Wv']).reshape(N, -1, NHEAD, DH).transpose(0, 2, 1, 3)
    logits = jnp.einsum('nhqd,nhkd->nhqk', qh, kh) / np.sqrt(DH)
    a = jax.nn.softmax(logits, axis=-1)
    o = jnp.einsum('nhqk,nhkd->nhqd', a, vh).transpose(0, 2, 1, 3).reshape(N, 1, D)
    o = o @ p['fus_Wo'] + p['fus_bo']
    return _layer_norm(q + o, p['fus_ln_g'], p['fus_ln_b'])


def kernel(q_t, f4, f8, f16, f32, c_t, params):
    N_t = q_t.shape[1]
    top_k_idx = _topk(c_t[0]).reshape(1, N_t, K)
    wcol = (top_k_idx % WF).astype(jnp.float32)
    hrow = (top_k_idx // WF).astype(jnp.float32)
    p_patch = jnp.stack([wcol, hrow], axis=-1) * STRIDE + STRIDE / 2.0
    norm = jnp.clip(p_patch / jnp.array([W_IMG, H_IMG], jnp.float32), 0.0, 1.0)
    ref = jnp.broadcast_to(norm.reshape(N_t * K, 1, 2), (N_t * K, NUM_LEVEL, 2))
    f_scales = jnp.concatenate([f4, f8, f16, f32], axis=1)[0]
    q_top_k = jnp.broadcast_to(q_t[0][:, None, :], (N_t, K, D)).reshape(N_t * K, D)
    for lp in params['layers']:
        q_top_k = _dmsmha_block(lp, q_top_k, f_scales, ref)
    q_top_k = q_top_k.reshape(1, N_t, K, D)
    q_cat = jnp.concatenate([q_top_k, jnp.broadcast_to(q_t[:, :, None, :], (1, N_t, K, D))], axis=-1)
    q_top_k = q_cat @ params['W_fusion'] + params['b_fusion']
    u_logit = q_top_k @ params['w_cert'] + params['b_cert']
    s_logit = q_top_k @ params['w_score'] + params['b_score']
    q_tk = q_top_k.reshape(N_t, K, D)
    q_init = q_t.reshape(N_t, 1, D)
    q_f = _mha_block(params, q_init, q_tk, q_tk)
    q_f = jnp.concatenate([q_f, q_init], axis=-1) @ params['W_final'] + params['b_final']
    return (q_f.reshape(1, N_t, D), p_patch, u_logit, s_logit)


# speedup vs baseline: 7.3892x; 7.3356x over previous
import functools

import jax
import jax.numpy as jnp
import numpy as np
from jax.experimental import pallas as pl
from jax.experimental.pallas import tpu as pltpu

K = 32
D = 128
NHEAD = 8
DH = D // NHEAD
H_IMG, W_IMG = 512, 512
STRIDE = 4
HF, WF = H_IMG // STRIDE, W_IMG // STRIDE
P = HF * WF
NUM_LEVEL = 4
N_POINTS = 4
N_LAYERS = 2
N_T = 512
DFF = 2 * D
SHAPES = [(HF, WF), (HF // 2, WF // 2), (HF // 4, WF // 4), (HF // 8, WF // 8)]
STARTS = [0, P, P + P // 4, P + P // 4 + P // 16]

ROWS_PER_BLK = 8


def _topk_body(c_ref, idx_ref, scratch):
    scratch[...] = c_ref[...]
    iota = jax.lax.broadcasted_iota(jnp.int32, (ROWS_PER_BLK, P), 1)
    kiota = jax.lax.broadcasted_iota(jnp.int32, (ROWS_PER_BLK, K), 1)

    def body(i, idxs):
        x = scratch[...]
        m = jnp.max(x, axis=1, keepdims=True)
        cand = jnp.where(x == m, iota, P)
        amin = jnp.min(cand, axis=1)
        scratch[...] = jnp.where(iota == amin[:, None], -jnp.inf, x)
        return jnp.where(kiota == i, amin[:, None], idxs)

    idxs = jax.lax.fori_loop(0, K, body, jnp.zeros((ROWS_PER_BLK, K), jnp.int32))
    idx_ref[0] = idxs


def _topk(c):
    nblk = N_T // ROWS_PER_BLK
    out = pl.pallas_call(
        _topk_body,
        grid=(nblk,),
        in_specs=[pl.BlockSpec((ROWS_PER_BLK, P), lambda i: (i, 0))],
        out_specs=pl.BlockSpec((1, ROWS_PER_BLK, K), lambda i: (i, 0, 0)),
        out_shape=jax.ShapeDtypeStruct((nblk, ROWS_PER_BLK, K), jnp.int32),
        scratch_shapes=[pltpu.VMEM((ROWS_PER_BLK, P), jnp.float32)],
    )(c)
    return out.reshape(N_T, K)


def _layer_norm(x, g, b):
    mu = jnp.mean(x, axis=-1, keepdims=True)
    var = jnp.var(x, axis=-1, keepdims=True)
    return (x - mu) / jnp.sqrt(var + 1e-5) * g + b


PAD_EXTRA = 136
P_PAD = STARTS[3] + SHAPES[3][0] * SHAPES[3][1] + PAD_EXTRA
TROWS = NHEAD * P_PAD
S_TOTAL = N_T * K * NHEAD * NUM_LEVEL * N_POINTS
GCHUNK = 128


def _sc_gather(table, idx_flat):
    from jax.experimental.pallas import tpu_sc as plsc
    info = plsc.get_sparse_core_info()
    ncores, nsub = info.num_cores, info.num_subcores
    nw = ncores * nsub
    per_w = S_TOTAL // nw
    nchunks = per_w // GCHUNK
    mesh = plsc.VectorSubcoreMesh(core_axis_name="c", subcore_axis_name="s")

    @functools.partial(
        pl.kernel, mesh=mesh,
        compiler_params=pltpu.CompilerParams(use_tc_tiling_on_sc=False),
        out_type=jax.ShapeDtypeStruct((S_TOTAL, 4 * DH), jnp.float32),
        scratch_types=[
            pltpu.VMEM((GCHUNK,), jnp.int32),
            pltpu.VMEM((GCHUNK, 4 * DH), jnp.float32),
            pltpu.SemaphoreType.DMA,
        ],
    )
    def gk(table_hbm, idx_hbm, out_hbm, idx_v, rows_v, sem):
        wid = jax.lax.axis_index("s") * ncores + jax.lax.axis_index("c")
        base = wid * per_w

        def body(j, carry):
            off = base + j * GCHUNK
            pltpu.sync_copy(idx_hbm.at[pl.ds(off, GCHUNK)], idx_v)
            pltpu.async_copy(table_hbm.at[idx_v], rows_v, sem).wait()
            pltpu.sync_copy(rows_v, out_hbm.at[pl.ds(off, GCHUNK)])
            return carry

        jax.lax.fori_loop(0, nchunks, body, 0)

    return gk(table, idx_flat)


def _build_table(v):
    vp = jnp.pad(v, ((0, PAD_EXTRA), (0, 0)))
    blocks = []
    for l in range(NUM_LEVEL):
        Hl, Wl = SHAPES[l]
        s = STARTS[l]
        n = Hl * Wl
        blk = jnp.stack(
            [jax.lax.dynamic_slice_in_dim(vp, s + d, n) for d in (0, 1, Wl, Wl + 1)],
            axis=1)
        blocks.append(blk)
    tab = jnp.concatenate(blocks, axis=0)
    tab = jnp.pad(tab, ((0, PAD_EXTRA), (0, 0), (0, 0)))
    tab = tab.reshape(P_PAD, 4, NHEAD, DH).transpose(2, 0, 1, 3)
    return tab.reshape(TROWS, 4 * DH)


def _coords(lp, q, ref):
    M = q.shape[0]
    off = (q @ lp['W_off'] + lp['b_off']).reshape(M, NHEAD, NUM_LEVEL, N_POINTS, 2)
    attn = (q @ lp['W_attn'] + lp['b_attn']).reshape(M, NHEAD, NUM_LEVEL * N_POINTS)
    attn = jax.nn.softmax(attn, axis=-1).reshape(M, NHEAD, NUM_LEVEL, N_POINTS)
    idx_list, w_list = [], []
    for l in range(NUM_LEVEL):
        Hl, Wl = SHAPES[l]
        s = STARTS[l]
        scale = jnp.array([Wl, Hl], jnp.float32)
        locs = ref[:, None, l, None, :] + off[:, :, l, :, :] / scale
        x = locs[..., 0] * Wl - 0.5
        y = locs[..., 1] * Hl - 0.5
        x0 = jnp.floor(x); y0 = jnp.floor(y)
        wx = x - x0; wy = y - y0
        x0i = jnp.clip(x0.astype(jnp.int32), 0, Wl - 1)
        y0i = jnp.clip(y0.astype(jnp.int32), 0, Hl - 1)
        foldx = x0i == Wl - 1
        foldy = y0i == Hl - 1
        w00 = (1 - wx) * (1 - wy); w01 = wx * (1 - wy)
        w10 = (1 - wx) * wy;       w11 = wx * wy
        z = jnp.zeros_like(wx)
        a00 = w00 + jnp.where(foldx, w01, z); a01 = jnp.where(foldx, z, w01)
        a10 = w10 + jnp.where(foldx, w11, z); a11 = jnp.where(foldx, z, w11)
        b00 = a00 + jnp.where(foldy, a10, z); b10 = jnp.where(foldy, z, a10)
        b01 = a01 + jnp.where(foldy, a11, z); b11 = jnp.where(foldy, z, a11)
        w4 = jnp.stack([b00, b01, b10, b11], -1) * attn[:, :, l, :, None]
        idx_list.append(y0i * Wl + x0i + s)
        w_list.append(w4)
    idx = jnp.stack(idx_list, 2)
    head_off = (jnp.arange(NHEAD, dtype=jnp.int32) * P_PAD)[None, :, None, None]
    idx_flat = (idx + head_off).reshape(S_TOTAL)
    w = jnp.stack(w_list, 2).reshape(M, NHEAD, NUM_LEVEL * N_POINTS, 4)
    return idx_flat, w


def _dmsmha_block(lp, q, kv, ref):
    M = q.shape[0]
    v = kv @ lp['W_val'] + lp['b_val']
    table = _build_table(v)
    idx_flat, w = _coords(lp, q, ref)
    rows = _sc_gather(table, idx_flat)
    sampled = rows.reshape(M, NHEAD, NUM_LEVEL * N_POINTS, 4, DH)
    out = jnp.sum(sampled * w[..., None], axis=(2, 3))
    out = out.reshape(M, D) @ lp['W_out'] + lp['b_out']
    q = _layer_norm(q + out, lp['ln1_g'], lp['ln1_b'])
    ff = jax.nn.relu(q @ lp['ffn_w1'] + lp['ffn_b1']) @ lp['ffn_w2'] + lp['ffn_b2']
    return _layer_norm(q + ff, lp['ln2_g'], lp['ln2_b'])


def _mha_block(p, q, k, v):
    N = q.shape[0]
    qh = (q @ p['fus_Wq']).reshape(N, 1, NHEAD, DH).transpose(0, 2, 1, 3)
    kh = (k @ p['fus_Wk']).reshape(N, -1, NHEAD, DH).transpose(0, 2, 1, 3)
    vh = (v @ p['fus_Wv']).reshape(N, -1, NHEAD, DH).transpose(0, 2, 1, 3)
    logits = jnp.einsum('nhqd,nhkd->nhqk', qh, kh) / np.sqrt(DH)
    a = jax.nn.softmax(logits, axis=-1)
    o = jnp.einsum('nhqk,nhkd->nhqd', a, vh).transpose(0, 2, 1, 3).reshape(N, 1, D)
    o = o @ p['fus_Wo'] + p['fus_bo']
    return _layer_norm(q + o, p['fus_ln_g'], p['fus_ln_b'])


def kernel(q_t, f4, f8, f16, f32, c_t, params):
    N_t = q_t.shape[1]
    top_k_idx = _topk(c_t[0]).reshape(1, N_t, K)
    wcol = (top_k_idx % WF).astype(jnp.float32)
    hrow = (top_k_idx // WF).astype(jnp.float32)
    p_patch = jnp.stack([wcol, hrow], axis=-1) * STRIDE + STRIDE / 2.0
    norm = jnp.clip(p_patch / jnp.array([W_IMG, H_IMG], jnp.float32), 0.0, 1.0)
    ref = jnp.broadcast_to(norm.reshape(N_t * K, 1, 2), (N_t * K, NUM_LEVEL, 2))
    f_scales = jnp.concatenate([f4, f8, f16, f32], axis=1)[0]
    q_top_k = jnp.broadcast_to(q_t[0][:, None, :], (N_t, K, D)).reshape(N_t * K, D)
    for lp in params['layers']:
        q_top_k = _dmsmha_block(lp, q_top_k, f_scales, ref)
    q_top_k = q_top_k.reshape(1, N_t, K, D)
    q_cat = jnp.concatenate([q_top_k, jnp.broadcast_to(q_t[:, :, None, :], (1, N_t, K, D))], axis=-1)
    q_top_k = q_cat @ params['W_fusion'] + params['b_fusion']
    u_logit = q_top_k @ params['w_cert'] + params['b_cert']
    s_logit = q_top_k @ params['w_score'] + params['b_score']
    q_tk = q_top_k.reshape(N_t, K, D)
    q_init = q_t.reshape(N_t, 1, D)
    q_f = _mha_block(params, q_init, q_tk, q_tk)
    q_f = jnp.concatenate([q_f, q_init], axis=-1) @ params['W_final'] + params['b_final']
    return (q_f.reshape(1, N_t, D), p_patch, u_logit, s_logit)


# speedup vs baseline: 16.5143x; 2.2349x over previous
import functools

import jax
import jax.numpy as jnp
import numpy as np
from jax.experimental import pallas as pl
from jax.experimental.pallas import tpu as pltpu

K = 32
D = 128
NHEAD = 8
DH = D // NHEAD
H_IMG, W_IMG = 512, 512
STRIDE = 4
HF, WF = H_IMG // STRIDE, W_IMG // STRIDE
P = HF * WF
NUM_LEVEL = 4
N_POINTS = 4
N_LAYERS = 2
N_T = 512
DFF = 2 * D
SHAPES = [(HF, WF), (HF // 2, WF // 2), (HF // 4, WF // 4), (HF // 8, WF // 8)]
STARTS = [0, P, P + P // 4, P + P // 4 + P // 16]

ROWS_PER_BLK = 8


def _topk_body(c_ref, idx_ref, scratch):
    scratch[...] = c_ref[...]
    iota = jax.lax.broadcasted_iota(jnp.int32, (ROWS_PER_BLK, P), 1)
    kiota = jax.lax.broadcasted_iota(jnp.int32, (ROWS_PER_BLK, K), 1)

    def body(i, idxs):
        x = scratch[...]
        m = jnp.max(x, axis=1, keepdims=True)
        cand = jnp.where(x == m, iota, P)
        amin = jnp.min(cand, axis=1)
        scratch[...] = jnp.where(iota == amin[:, None], -jnp.inf, x)
        return jnp.where(kiota == i, amin[:, None], idxs)

    idxs = jax.lax.fori_loop(0, K, body, jnp.zeros((ROWS_PER_BLK, K), jnp.int32))
    idx_ref[0] = idxs


def _topk(c):
    nblk = N_T // ROWS_PER_BLK
    out = pl.pallas_call(
        _topk_body,
        grid=(nblk,),
        in_specs=[pl.BlockSpec((ROWS_PER_BLK, P), lambda i: (i, 0))],
        out_specs=pl.BlockSpec((1, ROWS_PER_BLK, K), lambda i: (i, 0, 0)),
        out_shape=jax.ShapeDtypeStruct((nblk, ROWS_PER_BLK, K), jnp.int32),
        scratch_shapes=[pltpu.VMEM((ROWS_PER_BLK, P), jnp.float32)],
    )(c)
    return out.reshape(N_T, K)


def _layer_norm(x, g, b):
    mu = jnp.mean(x, axis=-1, keepdims=True)
    var = jnp.var(x, axis=-1, keepdims=True)
    return (x - mu) / jnp.sqrt(var + 1e-5) * g + b


PAD_EXTRA = 136
P_PAD = STARTS[3] + SHAPES[3][0] * SHAPES[3][1] + PAD_EXTRA
TROWS = NHEAD * P_PAD
S_TOTAL = N_T * K * NHEAD * NUM_LEVEL * N_POINTS
GCHUNK = 128


def _sc_gather(table, idx_flat):
    from jax.experimental.pallas import tpu_sc as plsc
    info = plsc.get_sparse_core_info()
    ncores, nsub = info.num_cores, info.num_subcores
    nw = ncores * nsub
    per_w = S_TOTAL // nw
    nchunks = per_w // GCHUNK
    mesh = plsc.VectorSubcoreMesh(core_axis_name="c", subcore_axis_name="s")

    @functools.partial(
        pl.kernel, mesh=mesh,
        out_type=jax.ShapeDtypeStruct((S_TOTAL, D), jnp.float32),
        scratch_types=[
            pltpu.VMEM((GCHUNK,), jnp.int32),
            pltpu.VMEM((GCHUNK, D), jnp.float32),
            pltpu.SemaphoreType.DMA,
        ],
    )
    def gk(table_hbm, idx_hbm, out_hbm, idx_v, rows_v, sem):
        wid = jax.lax.axis_index("s") * ncores + jax.lax.axis_index("c")
        base = wid * per_w

        def body(j, carry):
            off = base + j * GCHUNK
            pltpu.sync_copy(idx_hbm.at[pl.ds(off, GCHUNK)], idx_v)
            pltpu.async_copy(table_hbm.at[idx_v], rows_v, sem).wait()
            pltpu.sync_copy(rows_v, out_hbm.at[pl.ds(off, GCHUNK)])
            return carry

        jax.lax.fori_loop(0, nchunks, body, 0)

    return gk(table, idx_flat)


def _build_table(v):
    vp = jnp.pad(v, ((0, PAD_EXTRA), (0, 0)))
    blocks = []
    for l in range(NUM_LEVEL):
        Hl, Wl = SHAPES[l]
        s = STARTS[l]
        n = Hl * Wl
        blk = jnp.stack(
            [jax.lax.dynamic_slice_in_dim(vp, s + d, n) for d in (0, 1, Wl, Wl + 1)],
            axis=1)
        blocks.append(blk)
    tab = jnp.concatenate(blocks, axis=0)
    tab = jnp.pad(tab, ((0, PAD_EXTRA), (0, 0), (0, 0)))
    tab = tab.reshape(P_PAD, 4, NHEAD, DH).transpose(2, 0, 1, 3)
    tab = tab.reshape(TROWS, 4 * DH)
    return jnp.pad(tab, ((0, 0), (0, D - 4 * DH)))


def _coords(lp, q, ref):
    M = q.shape[0]
    off = (q @ lp['W_off'] + lp['b_off']).reshape(M, NHEAD, NUM_LEVEL, N_POINTS, 2)
    attn = (q @ lp['W_attn'] + lp['b_attn']).reshape(M, NHEAD, NUM_LEVEL * N_POINTS)
    attn = jax.nn.softmax(attn, axis=-1).reshape(M, NHEAD, NUM_LEVEL, N_POINTS)
    idx_list, w_list = [], []
    for l in range(NUM_LEVEL):
        Hl, Wl = SHAPES[l]
        s = STARTS[l]
        scale = jnp.array([Wl, Hl], jnp.float32)
        locs = ref[:, None, l, None, :] + off[:, :, l, :, :] / scale
        x = locs[..., 0] * Wl - 0.5
        y = locs[..., 1] * Hl - 0.5
        x0 = jnp.floor(x); y0 = jnp.floor(y)
        wx = x - x0; wy = y - y0
        x0i = jnp.clip(x0.astype(jnp.int32), 0, Wl - 1)
        y0i = jnp.clip(y0.astype(jnp.int32), 0, Hl - 1)
        foldx = x0i == Wl - 1
        foldy = y0i == Hl - 1
        w00 = (1 - wx) * (1 - wy); w01 = wx * (1 - wy)
        w10 = (1 - wx) * wy;       w11 = wx * wy
        z = jnp.zeros_like(wx)
        a00 = w00 + jnp.where(foldx, w01, z); a01 = jnp.where(foldx, z, w01)
        a10 = w10 + jnp.where(foldx, w11, z); a11 = jnp.where(foldx, z, w11)
        b00 = a00 + jnp.where(foldy, a10, z); b10 = jnp.where(foldy, z, a10)
        b01 = a01 + jnp.where(foldy, a11, z); b11 = jnp.where(foldy, z, a11)
        w4 = jnp.stack([b00, b01, b10, b11], -1) * attn[:, :, l, :, None]
        idx_list.append(y0i * Wl + x0i + s)
        w_list.append(w4)
    idx = jnp.stack(idx_list, 2)
    head_off = (jnp.arange(NHEAD, dtype=jnp.int32) * P_PAD)[None, :, None, None]
    idx_flat = (idx + head_off).reshape(S_TOTAL)
    w = jnp.stack(w_list, 2).reshape(M, NHEAD, NUM_LEVEL * N_POINTS, 4)
    return idx_flat, w


MBLK = 64
NS = NUM_LEVEL * N_POINTS


def _post_body(rows_ref, w_ref, q_ref, wout_ref, bout_ref, ln1g_ref, ln1b_ref,
               w1_ref, b1_ref, w2_ref, b2_ref, ln2g_ref, ln2b_ref, out_ref):
    rows = rows_ref[...]
    w4 = w_ref[...]
    acc = rows[:, :, 0:DH] * w4[:, :, 0][..., None]
    for c in range(1, 4):
        acc = acc + rows[:, :, c * DH:(c + 1) * DH] * w4[:, :, c][..., None]
    g = jnp.sum(acc.reshape(MBLK, NHEAD, NS, DH), axis=2)
    out = g.reshape(MBLK, D)
    y = out @ wout_ref[...] + bout_ref[...] + q_ref[...]
    y = _layer_norm(y, ln1g_ref[...], ln1b_ref[...])
    ff = jnp.maximum(y @ w1_ref[...] + b1_ref[...], 0.0) @ w2_ref[...] + b2_ref[...]
    out_ref[...] = _layer_norm(y + ff, ln2g_ref[...], ln2b_ref[...])


def _post(rows, w, q, lp):
    M = q.shape[0]
    grid = (M // MBLK,)
    full = lambda shape: pl.BlockSpec(shape, lambda i: tuple(0 for _ in shape))
    return pl.pallas_call(
        _post_body,
        grid=grid,
        in_specs=[
            pl.BlockSpec((MBLK, NHEAD * NS, D), lambda i: (i, 0, 0)),
            pl.BlockSpec((MBLK, NHEAD * NS, 4), lambda i: (i, 0, 0)),
            pl.BlockSpec((MBLK, D), lambda i: (i, 0)),
            full((D, D)), full((1, D)), full((1, D)), full((1, D)),
            full((D, DFF)), full((1, DFF)), full((DFF, D)), full((1, D)),
            full((1, D)), full((1, D)),
        ],
        out_specs=pl.BlockSpec((MBLK, D), lambda i: (i, 0)),
        out_shape=jax.ShapeDtypeStruct((M, D), jnp.float32),
    )(rows, w, q, lp['W_out'], lp['b_out'].reshape(1, D),
      lp['ln1_g'].reshape(1, D), lp['ln1_b'].reshape(1, D),
      lp['ffn_w1'], lp['ffn_b1'].reshape(1, DFF), lp['ffn_w2'],
      lp['ffn_b2'].reshape(1, D), lp['ln2_g'].reshape(1, D),
      lp['ln2_b'].reshape(1, D))


def _dmsmha_block(lp, q, kv, ref):
    M = q.shape[0]
    v = kv @ lp['W_val'] + lp['b_val']
    table = _build_table(v)
    idx_flat, w = _coords(lp, q, ref)
    rows = _sc_gather(table, idx_flat)
    rows = rows.reshape(M, NHEAD * NS, D)
    w = w.reshape(M, NHEAD * NS, 4)
    return _post(rows, w, q, lp)


def _mha_block(p, q, k, v):
    N = q.shape[0]
    qh = (q @ p['fus_Wq']).reshape(N, 1, NHEAD, DH).transpose(0, 2, 1, 3)
    kh = (k @ p['fus_Wk']).reshape(N, -1, NHEAD, DH).transpose(0, 2, 1, 3)
    vh = (v @ p['fus_Wv']).reshape(N, -1, NHEAD, DH).transpose(0, 2, 1, 3)
    logits = jnp.einsum('nhqd,nhkd->nhqk', qh, kh) / np.sqrt(DH)
    a = jax.nn.softmax(logits, axis=-1)
    o = jnp.einsum('nhqk,nhkd->nhqd', a, vh).transpose(0, 2, 1, 3).reshape(N, 1, D)
    o = o @ p['fus_Wo'] + p['fus_bo']
    return _layer_norm(q + o, p['fus_ln_g'], p['fus_ln_b'])


def kernel(q_t, f4, f8, f16, f32, c_t, params):
    N_t = q_t.shape[1]
    top_k_idx = _topk(c_t[0]).reshape(1, N_t, K)
    wcol = (top_k_idx % WF).astype(jnp.float32)
    hrow = (top_k_idx // WF).astype(jnp.float32)
    p_patch = jnp.stack([wcol, hrow], axis=-1) * STRIDE + STRIDE / 2.0
    norm = jnp.clip(p_patch / jnp.array([W_IMG, H_IMG], jnp.float32), 0.0, 1.0)
    ref = jnp.broadcast_to(norm.reshape(N_t * K, 1, 2), (N_t * K, NUM_LEVEL, 2))
    f_scales = jnp.concatenate([f4, f8, f16, f32], axis=1)[0]
    q_top_k = jnp.broadcast_to(q_t[0][:, None, :], (N_t, K, D)).reshape(N_t * K, D)
    for lp in params['layers']:
        q_top_k = _dmsmha_block(lp, q_top_k, f_scales, ref)
    q_top_k = q_top_k.reshape(1, N_t, K, D)
    q_cat = jnp.concatenate([q_top_k, jnp.broadcast_to(q_t[:, :, None, :], (1, N_t, K, D))], axis=-1)
    q_top_k = q_cat @ params['W_fusion'] + params['b_fusion']
    u_logit = q_top_k @ params['w_cert'] + params['b_cert']
    s_logit = q_top_k @ params['w_score'] + params['b_score']
    q_tk = q_top_k.reshape(N_t, K, D)
    q_init = q_t.reshape(N_t, 1, D)
    q_f = _mha_block(params, q_init, q_tk, q_tk)
    q_f = jnp.concatenate([q_f, q_init], axis=-1) @ params['W_final'] + params['b_final']
    return (q_f.reshape(1, N_t, D), p_patch, u_logit, s_logit)


# speedup vs baseline: 25.4511x; 1.5412x over previous
import functools

import jax
import jax.numpy as jnp
import numpy as np
from jax.experimental import pallas as pl
from jax.experimental.pallas import tpu as pltpu

K = 32
D = 128
NHEAD = 8
DH = D // NHEAD
H_IMG, W_IMG = 512, 512
STRIDE = 4
HF, WF = H_IMG // STRIDE, W_IMG // STRIDE
P = HF * WF
NUM_LEVEL = 4
N_POINTS = 4
N_LAYERS = 2
N_T = 512
DFF = 2 * D
SHAPES = [(HF, WF), (HF // 2, WF // 2), (HF // 4, WF // 4), (HF // 8, WF // 8)]
STARTS = [0, P, P + P // 4, P + P // 4 + P // 16]

ROWS_PER_BLK = 8


def _topk_body(c_ref, idx_ref, scratch):
    scratch[...] = c_ref[...]
    iota = jax.lax.broadcasted_iota(jnp.int32, (ROWS_PER_BLK, P), 1)
    kiota = jax.lax.broadcasted_iota(jnp.int32, (ROWS_PER_BLK, K), 1)

    def body(i, idxs):
        x = scratch[...]
        m = jnp.max(x, axis=1, keepdims=True)
        cand = jnp.where(x == m, iota, P)
        amin = jnp.min(cand, axis=1)
        scratch[...] = jnp.where(iota == amin[:, None], -jnp.inf, x)
        return jnp.where(kiota == i, amin[:, None], idxs)

    idxs = jax.lax.fori_loop(0, K, body, jnp.zeros((ROWS_PER_BLK, K), jnp.int32))
    idx_ref[0] = idxs


def _topk(c):
    nblk = N_T // ROWS_PER_BLK
    out = pl.pallas_call(
        _topk_body,
        grid=(nblk,),
        in_specs=[pl.BlockSpec((ROWS_PER_BLK, P), lambda i: (i, 0))],
        out_specs=pl.BlockSpec((1, ROWS_PER_BLK, K), lambda i: (i, 0, 0)),
        out_shape=jax.ShapeDtypeStruct((nblk, ROWS_PER_BLK, K), jnp.int32),
        scratch_shapes=[pltpu.VMEM((ROWS_PER_BLK, P), jnp.float32)],
    )(c)
    return out.reshape(N_T, K)


def _layer_norm(x, g, b):
    mu = jnp.mean(x, axis=-1, keepdims=True)
    var = jnp.var(x, axis=-1, keepdims=True)
    return (x - mu) / jnp.sqrt(var + 1e-5) * g + b


PAD_EXTRA = 136
P_PAD = STARTS[3] + SHAPES[3][0] * SHAPES[3][1] + PAD_EXTRA
TROWS = NHEAD * P_PAD
S_TOTAL = N_T * K * NHEAD * NUM_LEVEL * N_POINTS
GCHUNK = 128


def _sc_gather(table, idx_flat):
    from jax.experimental.pallas import tpu_sc as plsc
    info = plsc.get_sparse_core_info()
    ncores, nsub = info.num_cores, info.num_subcores
    nw = ncores * nsub
    per_w = S_TOTAL // nw
    nchunks = per_w // GCHUNK
    mesh = plsc.VectorSubcoreMesh(core_axis_name="c", subcore_axis_name="s")

    @functools.partial(
        pl.kernel, mesh=mesh,
        out_type=jax.ShapeDtypeStruct((S_TOTAL, D), jnp.float32),
        scratch_types=[
            pltpu.VMEM((GCHUNK,), jnp.int32),
            pltpu.VMEM((GCHUNK, D), jnp.float32),
            pltpu.SemaphoreType.DMA,
        ],
    )
    def gk(table_hbm, idx_hbm, out_hbm, idx_v, rows_v, sem):
        wid = jax.lax.axis_index("s") * ncores + jax.lax.axis_index("c")
        base = wid * per_w

        def body(j, carry):
            off = base + j * GCHUNK
            pltpu.sync_copy(idx_hbm.at[pl.ds(off, GCHUNK)], idx_v)
            pltpu.async_copy(table_hbm.at[idx_v], rows_v, sem).wait()
            pltpu.sync_copy(rows_v, out_hbm.at[pl.ds(off, GCHUNK)])
            return carry

        jax.lax.fori_loop(0, nchunks, body, 0)

    return gk(table, idx_flat)


def _lane_consts():
    lane = np.arange(D)
    lvl = (lane % (NUM_LEVEL * N_POINTS)) // N_POINTS
    head = lane // (NUM_LEVEL * N_POINTS)
    Wl = np.array([SHAPES[l][1] for l in range(NUM_LEVEL)])[lvl]
    Hl = np.array([SHAPES[l][0] for l in range(NUM_LEVEL)])[lvl]
    sv = np.array(STARTS)[lvl]
    hoff = head * P_PAD
    return (jnp.asarray(Wl[None], jnp.float32), jnp.asarray(Hl[None], jnp.float32),
            jnp.asarray(Wl[None], jnp.int32), jnp.asarray(Hl[None], jnp.int32),
            jnp.asarray((sv + hoff)[None], jnp.int32))


_WLF, _HLF, _WLI, _HLI, _SOFF = None, None, None, None, None


def _off_perm():
    j = np.arange(D)
    return np.concatenate([2 * j, 2 * j + 1])


def _coords_body(q_ref, ref_ref, wo_ref, bo_ref, wa_ref, ba_ref,
                 wlf_ref, hlf_ref, wli_ref, hli_ref, soff_ref,
                 idx_ref, w0_ref, w1_ref, w2_ref, w3_ref):
    q = q_ref[...]
    proj = q @ wo_ref[...] + bo_ref[...]
    offx = proj[:, 0:D]
    offy = proj[:, D:2 * D]
    za = q @ wa_ref[...] + ba_ref[...]
    blk = za.shape[0]
    z3 = za.reshape(blk, NHEAD, NUM_LEVEL * N_POINTS)
    z3 = z3 - jnp.max(z3, axis=-1, keepdims=True)
    e3 = jnp.exp(z3)
    attnw = (e3 / jnp.sum(e3, axis=-1, keepdims=True)).reshape(blk, D)
    wlf = wlf_ref[...]; hlf = hlf_ref[...]
    refx = ref_ref[:, 0:1]; refy = ref_ref[:, 1:2]
    x = (refx + offx / wlf) * wlf - 0.5
    y = (refy + offy / hlf) * hlf - 0.5
    x0f = jnp.floor(x); y0f = jnp.floor(y)
    wx = x - x0f; wy = y - y0f
    wli = wli_ref[...]; hli = hli_ref[...]
    x0i = jnp.clip(x0f.astype(jnp.int32), 0, wli - 1)
    y0i = jnp.clip(y0f.astype(jnp.int32), 0, hli - 1)
    foldx = x0i == wli - 1
    foldy = y0i == hli - 1
    w00 = (1 - wx) * (1 - wy); w01 = wx * (1 - wy)
    w10 = (1 - wx) * wy;       w11 = wx * wy
    z = jnp.zeros_like(wx)
    a00 = w00 + jnp.where(foldx, w01, z); a01 = jnp.where(foldx, z, w01)
    a10 = w10 + jnp.where(foldx, w11, z); a11 = jnp.where(foldx, z, w11)
    b00 = a00 + jnp.where(foldy, a10, z); b10 = jnp.where(foldy, z, a10)
    b01 = a01 + jnp.where(foldy, a11, z); b11 = jnp.where(foldy, z, a11)
    idx_ref[...] = y0i * wli + x0i + soff_ref[...]
    w0_ref[...] = b00 * attnw
    w1_ref[...] = b01 * attnw
    w2_ref[...] = b10 * attnw
    w3_ref[...] = b11 * attnw


CBLK = 512


def _coords(lp, q, refxy):
    M = q.shape[0]
    wo = lp['W_off'][:, _off_perm()]
    bo = lp['b_off'][_off_perm()].reshape(1, 2 * D)
    consts = _lane_consts()
    full = lambda shape: pl.BlockSpec(shape, lambda i: tuple(0 for _ in shape))
    mspec = pl.BlockSpec((CBLK, D), lambda i: (i, 0))
    out = pl.pallas_call(
        _coords_body,
        grid=(M // CBLK,),
        in_specs=[
            mspec, pl.BlockSpec((CBLK, 2), lambda i: (i, 0)),
            full((D, 2 * D)), full((1, 2 * D)), full((D, D)), full((1, D)),
            full((1, D)), full((1, D)), full((1, D)), full((1, D)), full((1, D)),
        ],
        out_specs=[mspec] * 5,
        out_shape=[jax.ShapeDtypeStruct((M, D), jnp.int32)]
        + [jax.ShapeDtypeStruct((M, D), jnp.float32)] * 4,
    )(q, refxy, wo, bo, lp['W_attn'], lp['b_attn'].reshape(1, D), *consts)
    return out[0].reshape(S_TOTAL), out[1], out[2], out[3], out[4]


def _sel_mats():
    E = np.zeros((NHEAD, 4, D, D), np.float32)
    for h in range(NHEAD):
        for c in range(4):
            for dh in range(DH):
                E[h, c, h * DH + dh, c * DH + dh] = 1.0
    return jnp.asarray(E)


VBLK = 640
ABLK = 952


def _vmat_body(f_ref, w_ref, b_ref, o_ref):
    o_ref[...] = f_ref[...] @ w_ref[...] + b_ref[...]


def _asm_body(v0_ref, v1_ref, v2_ref, v3_ref, e_ref, o_ref):
    e = e_ref[0]
    acc = v0_ref[...] @ e[0]
    acc = acc + v1_ref[...] @ e[1]
    acc = acc + v2_ref[...] @ e[2]
    acc = acc + v3_ref[...] @ e[3]
    o_ref[0] = acc


def _build_table(kv, lp, emats):
    P_total = kv.shape[0]
    full = lambda shape: pl.BlockSpec(shape, lambda *a: tuple(0 for _ in shape))
    v = pl.pallas_call(
        _vmat_body,
        grid=(P_total // VBLK,),
        in_specs=[pl.BlockSpec((VBLK, D), lambda i: (i, 0)),
                  full((D, D)), full((1, D))],
        out_specs=pl.BlockSpec((VBLK, D), lambda i: (i, 0)),
        out_shape=jax.ShapeDtypeStruct((P_total, D), jnp.float32),
    )(kv, lp['W_val'], lp['b_val'].reshape(1, D))
    vp = jnp.pad(v, ((0, P_PAD - P_total), (0, 0)))
    vsh = []
    for c in range(4):
        parts = []
        for l in range(NUM_LEVEL):
            Hl, Wl = SHAPES[l]
            d = (0, 1, Wl, Wl + 1)[c]
            parts.append(jax.lax.dynamic_slice_in_dim(vp, STARTS[l] + d, Hl * Wl))
        sh = jnp.concatenate(parts, axis=0)
        vsh.append(jnp.pad(sh, ((0, P_PAD - P_total), (0, 0))))
    tab = pl.pallas_call(
        _asm_body,
        grid=(NHEAD, P_PAD // ABLK),
        in_specs=[pl.BlockSpec((ABLK, D), lambda h, j: (j, 0))] * 4
        + [pl.BlockSpec((1, 4, D, D), lambda h, j: (h, 0, 0, 0))],
        out_specs=pl.BlockSpec((1, ABLK, D), lambda h, j: (h, j, 0)),
        out_shape=jax.ShapeDtypeStruct((NHEAD, P_PAD, D), jnp.float32),
    )(*vsh, emats)
    return tab.reshape(TROWS, D)


MBLK = 64
NS = NUM_LEVEL * N_POINTS


def _post_body(rows_ref, w0_ref, w1_ref, w2_ref, w3_ref, q_ref, wout_ref,
               bout_ref, ln1g_ref, ln1b_ref,
               w1f_ref, b1f_ref, w2f_ref, b2f_ref, ln2g_ref, ln2b_ref, out_ref):
    rows = rows_ref[...]
    ws = (w0_ref, w1_ref, w2_ref, w3_ref)
    acc = rows[:, :, 0:DH] * w0_ref[...][..., None]
    for c in range(1, 4):
        acc = acc + rows[:, :, c * DH:(c + 1) * DH] * ws[c][...][..., None]
    g = jnp.sum(acc.reshape(MBLK, NHEAD, NS, DH), axis=2)
    out = g.reshape(MBLK, D)
    y = out @ wout_ref[...] + bout_ref[...] + q_ref[...]
    y = _layer_norm(y, ln1g_ref[...], ln1b_ref[...])
    ff = jnp.maximum(y @ w1f_ref[...] + b1f_ref[...], 0.0) @ w2f_ref[...] + b2f_ref[...]
    out_ref[...] = _layer_norm(y + ff, ln2g_ref[...], ln2b_ref[...])


def _post(rows, w0, w1, w2, w3, q, lp):
    M = q.shape[0]
    grid = (M // MBLK,)
    full = lambda shape: pl.BlockSpec(shape, lambda i: tuple(0 for _ in shape))
    mspec = pl.BlockSpec((MBLK, D), lambda i: (i, 0))
    return pl.pallas_call(
        _post_body,
        grid=grid,
        in_specs=[
            pl.BlockSpec((MBLK, NHEAD * NS, D), lambda i: (i, 0, 0)),
            mspec, mspec, mspec, mspec, mspec,
            full((D, D)), full((1, D)), full((1, D)), full((1, D)),
            full((D, DFF)), full((1, DFF)), full((DFF, D)), full((1, D)),
            full((1, D)), full((1, D)),
        ],
        out_specs=mspec,
        out_shape=jax.ShapeDtypeStruct((M, D), jnp.float32),
    )(rows, w0, w1, w2, w3, q, lp['W_out'], lp['b_out'].reshape(1, D),
      lp['ln1_g'].reshape(1, D), lp['ln1_b'].reshape(1, D),
      lp['ffn_w1'], lp['ffn_b1'].reshape(1, DFF), lp['ffn_w2'],
      lp['ffn_b2'].reshape(1, D), lp['ln2_g'].reshape(1, D),
      lp['ln2_b'].reshape(1, D))


def _dmsmha_block(lp, q, kv, refxy, emats):
    M = q.shape[0]
    table = _build_table(kv, lp, emats)
    idx_flat, w0, w1, w2, w3 = _coords(lp, q, refxy)
    rows = _sc_gather(table, idx_flat)
    rows = rows.reshape(M, NHEAD * NS, D)
    return _post(rows, w0, w1, w2, w3, q, lp)


def _mha_block(p, q, k, v):
    N = q.shape[0]
    qh = (q @ p['fus_Wq']).reshape(N, 1, NHEAD, DH).transpose(0, 2, 1, 3)
    kh = (k @ p['fus_Wk']).reshape(N, -1, NHEAD, DH).transpose(0, 2, 1, 3)
    vh = (v @ p['fus_Wv']).reshape(N, -1, NHEAD, DH).transpose(0, 2, 1, 3)
    logits = jnp.einsum('nhqd,nhkd->nhqk', qh, kh) / np.sqrt(DH)
    a = jax.nn.softmax(logits, axis=-1)
    o = jnp.einsum('nhqk,nhkd->nhqd', a, vh).transpose(0, 2, 1, 3).reshape(N, 1, D)
    o = o @ p['fus_Wo'] + p['fus_bo']
    return _layer_norm(q + o, p['fus_ln_g'], p['fus_ln_b'])


def kernel(q_t, f4, f8, f16, f32, c_t, params):
    N_t = q_t.shape[1]
    top_k_idx = _topk(c_t[0]).reshape(1, N_t, K)
    wcol = (top_k_idx % WF).astype(jnp.float32)
    hrow = (top_k_idx // WF).astype(jnp.float32)
    p_patch = jnp.stack([wcol, hrow], axis=-1) * STRIDE + STRIDE / 2.0
    norm = jnp.clip(p_patch / jnp.array([W_IMG, H_IMG], jnp.float32), 0.0, 1.0)
    refxy = norm.reshape(N_t * K, 2)
    f_scales = jnp.concatenate([f4, f8, f16, f32], axis=1)[0]
    q_top_k = jnp.broadcast_to(q_t[0][:, None, :], (N_t, K, D)).reshape(N_t * K, D)
    emats = _sel_mats()
    for lp in params['layers']:
        q_top_k = _dmsmha_block(lp, q_top_k, f_scales, refxy, emats)
    q_top_k = q_top_k.reshape(1, N_t, K, D)
    q_cat = jnp.concatenate([q_top_k, jnp.broadcast_to(q_t[:, :, None, :], (1, N_t, K, D))], axis=-1)
    q_top_k = q_cat @ params['W_fusion'] + params['b_fusion']
    u_logit = q_top_k @ params['w_cert'] + params['b_cert']
    s_logit = q_top_k @ params['w_score'] + params['b_score']
    q_tk = q_top_k.reshape(N_t, K, D)
    q_init = q_t.reshape(N_t, 1, D)
    q_f = _mha_block(params, q_init, q_tk, q_tk)
    q_f = jnp.concatenate([q_f, q_init], axis=-1) @ params['W_final'] + params['b_final']
    return (q_f.reshape(1, N_t, D), p_patch, u_logit, s_logit)
